# baseline (speedup 1.0000x reference)
import functools

import jax
import jax.numpy as jnp
from jax import lax
from jax.experimental import pallas as pl
from jax.experimental.pallas import tpu as pltpu
from jax.experimental.pallas import tpu_sc as plsc

NN = 4


def _topk_body(num_steps, tgt_ref, seq_ref, idx_out_ref, vals_ref, gidx_ref):
    step = pl.program_id(0)
    sb, b, d = seq_ref.shape

    @pl.when(step == 0)
    def _init():
        vals_ref[...] = jnp.full((NN, b), -jnp.inf, jnp.float32)
        gidx_ref[...] = jnp.zeros((NN, b), jnp.int32)

    seq = seq_ref[...]
    tgt = tgt_ref[...]
    scores = jnp.sum(seq * tgt[None], axis=-1)
    rowid = step * sb + lax.broadcasted_iota(jnp.int32, (sb, b), 0)

    x = jnp.concatenate([vals_ref[...], scores], axis=0)
    xi = jnp.concatenate([gidx_ref[...], rowid], axis=0)

    new_vals = []
    new_idx = []
    big = jnp.int32(2**30)
    for _ in range(NN):
        m = jnp.max(x, axis=0)
        sel = jnp.min(jnp.where(x == m[None], xi, big), axis=0)
        x = jnp.where(xi == sel[None], -jnp.inf, x)
        new_vals.append(m)
        new_idx.append(sel)
    vals_ref[...] = jnp.stack(new_vals, axis=0)
    gidx_ref[...] = jnp.stack(new_idx, axis=0)

    @pl.when(step == num_steps - 1)
    def _fin():
        col = lax.broadcasted_iota(jnp.int32, (NN, b), 1)
        idx_out_ref[...] = gidx_ref[...] * b + col


def _topk_indices(target_vec, seq_vecs, block_s=32):
    S, B, D = seq_vecs.shape
    num_steps = S // block_s
    return pl.pallas_call(
        functools.partial(_topk_body, num_steps),
        grid=(num_steps,),
        in_specs=[
            pl.BlockSpec((B, D), lambda i: (0, 0)),
            pl.BlockSpec((block_s, B, D), lambda i: (i, 0, 0)),
        ],
        out_specs=pl.BlockSpec((NN, B), lambda i: (0, 0)),
        out_shape=jax.ShapeDtypeStruct((NN, B), jnp.int32),
        scratch_shapes=[
            pltpu.VMEM((NN, B), jnp.float32),
            pltpu.VMEM((NN, B), jnp.int32),
        ],
    )(target_vec, seq_vecs)


def _sc_gather(table, flat_idx, n_rows, d):
    info = plsc.get_sparse_core_info()
    nw = info.num_cores * info.num_subcores
    per_w = n_rows // nw
    mesh = plsc.VectorSubcoreMesh(core_axis_name="c", subcore_axis_name="s")

    @functools.partial(
        pl.kernel,
        out_type=jax.ShapeDtypeStruct((n_rows, d), jnp.float32),
        mesh=mesh,
        scratch_types=[
            pltpu.VMEM((per_w,), jnp.int32),
            pltpu.VMEM((per_w, d), jnp.float32),
            pltpu.SemaphoreType.DMA,
        ],
    )
    def gather_kernel(table_hbm, idx_hbm, out_hbm, idx_v, rows_v, sem):
        wid = lax.axis_index("s") * info.num_cores + lax.axis_index("c")
        base = wid * per_w
        pltpu.sync_copy(idx_hbm.at[pl.ds(base, per_w)], idx_v)
        pltpu.async_copy(table_hbm.at[idx_v], rows_v, sem).wait()
        pltpu.sync_copy(rows_v, out_hbm.at[pl.ds(base, per_w)])

    return gather_kernel(table, flat_idx)


def kernel(target_vec, seq_vecs):
    S, B, D = seq_vecs.shape
    flat_idx = _topk_indices(target_vec, seq_vecs).reshape(-1)
    flat = seq_vecs.reshape(S * B, D)
    rows = _sc_gather(flat, flat_idx, NN * B, D)
    return rows.reshape(NN, B, D)


# speedup vs baseline: 1.0139x; 1.0139x over previous
import functools

import jax
import jax.numpy as jnp
from jax import lax
from jax.experimental import pallas as pl
from jax.experimental.pallas import tpu as pltpu
from jax.experimental.pallas import tpu_sc as plsc

NN = 4
S_SC = 512
CH = 64


def _merge_top4(x, xi):
    nrows, b = x.shape
    big = jnp.int32(2**30)
    vals, idx = [], []
    for _ in range(NN):
        m = jnp.max(x, axis=0)
        sel = jnp.min(jnp.where(x == m[None], xi, big), axis=0)
        x = jnp.where(xi == sel[None], -jnp.inf, x)
        vals.append(m)
        idx.append(sel)
    return jnp.stack(vals, axis=0), jnp.stack(idx, axis=0)


def _tc_topk_body(num_steps, tgt_ref, seq_ref, vals_ref, sidx_ref):
    step = pl.program_id(0)
    sb, b, d = seq_ref.shape

    @pl.when(step == 0)
    def _init():
        vals_ref[...] = jnp.full((NN, b), -jnp.inf, jnp.float32)
        sidx_ref[...] = jnp.zeros((NN, b), jnp.int32)

    seq = seq_ref[...]
    tgt = tgt_ref[...]
    scores = jnp.sum(seq * tgt[None], axis=-1)
    rowid = step * sb + lax.broadcasted_iota(jnp.int32, (sb, b), 0)
    x = jnp.concatenate([vals_ref[...], scores], axis=0)
    xi = jnp.concatenate([sidx_ref[...], rowid], axis=0)
    nv, ni = _merge_top4(x, xi)
    vals_ref[...] = nv
    sidx_ref[...] = ni


def _tc_topk(target_vec, seq_all, s_tc, block_s=64):
    S, B, D = seq_all.shape
    num_steps = s_tc // block_s
    return pl.pallas_call(
        functools.partial(_tc_topk_body, num_steps),
        grid=(num_steps,),
        in_specs=[
            pl.BlockSpec((B, D), lambda i: (0, 0)),
            pl.BlockSpec((block_s, B, D), lambda i: (i, 0, 0)),
        ],
        out_specs=[
            pl.BlockSpec((NN, B), lambda i: (0, 0)),
            pl.BlockSpec((NN, B), lambda i: (0, 0)),
        ],
        out_shape=[
            jax.ShapeDtypeStruct((NN, B), jnp.float32),
            jax.ShapeDtypeStruct((NN, B), jnp.int32),
        ],
    )(target_vec, seq_all)


def _sc_scores(target_vec, seq_flat, s_lo, s_sc, B, D):
    info = plsc.get_sparse_core_info()
    nw = info.num_cores * info.num_subcores
    b_per_w = B // nw
    n_ch = s_sc // CH
    nd = D // 16
    mesh = plsc.VectorSubcoreMesh(core_axis_name="c", subcore_axis_name="s")

    @functools.partial(
        pl.kernel,
        out_type=jax.ShapeDtypeStruct((B, s_sc), jnp.float32),
        mesh=mesh,
        scratch_types=[
            pltpu.VMEM((CH, D), jnp.float32),
            pltpu.VMEM((CH, D), jnp.float32),
            pltpu.VMEM((b_per_w, D), jnp.float32),
            pltpu.VMEM((CH,), jnp.int32),
            pltpu.VMEM((CH,), jnp.int32),
            pltpu.VMEM((b_per_w, s_sc), jnp.float32),
            pltpu.SemaphoreType.DMA,
            pltpu.SemaphoreType.DMA,
        ],
    )
    def scorer(tgt_hbm, seq_hbm, out_hbm, buf0, buf1, tgt_v, idx0, idx1,
               sc_v, sem0, sem1):
        wid = lax.axis_index("s") * info.num_cores + lax.axis_index("c")
        base_b = wid * b_per_w
        pltpu.sync_copy(tgt_hbm.at[pl.ds(base_b, b_per_w)], tgt_v)
        lane = lax.broadcasted_iota(jnp.int32, (16,), 0)
        dnums = lax.GatherDimensionNumbers(
            offset_dims=(), collapsed_slice_dims=(0,), start_index_map=(0,))

        def shuf(v, sh):
            perm = jnp.bitwise_xor(lane, sh)
            return lax.gather(v, perm[:, None], dnums, slice_sizes=(1,),
                              mode=lax.GatherScatterMode.PROMISE_IN_BOUNDS)

        def combine(a, b, sh):
            return jnp.where((lane & sh) == 0, a + shuf(a, sh), b + shuf(b, sh))

        def compute_chunk(buf, b_loc, out_base):
            def group(g, _):
                tr = [tgt_v[b_loc, pl.ds(16 * j, 16)] for j in range(nd)]
                accs = []
                for l in range(16):
                    row = g * 16 + l
                    acc = [jnp.zeros((16,), jnp.float32) for _ in range(4)]
                    for j in range(nd):
                        acc[j % 4] = acc[j % 4] + buf[row, pl.ds(16 * j, 16)] * tr[j]
                    accs.append((acc[0] + acc[1]) + (acc[2] + acc[3]))
                sh = 1
                while len(accs) > 1:
                    accs = [combine(accs[2 * i], accs[2 * i + 1], sh)
                            for i in range(len(accs) // 2)]
                    sh *= 2
                sc_v[b_loc, pl.ds(out_base + g * 16, 16)] = accs[0]
                return 0
            lax.fori_loop(0, CH // 16, group, 0, unroll=False)

        def set_idx(idx_ref, ch, b):
            for q in range(CH // 16):
                idx_ref[pl.ds(16 * q, 16)] = (
                    (s_lo + ch * CH + 16 * q + lane) * B + b)

        def bump_idx(idx_ref):
            for q in range(CH // 16):
                idx_ref[pl.ds(16 * q, 16)] = idx_ref[pl.ds(16 * q, 16)] + 2 * CH * B

        for b_loc in range(b_per_w):
            b = base_b + b_loc
            set_idx(idx0, 0, b)
            set_idx(idx1, 1, b)
            pltpu.make_async_copy(seq_hbm.at[idx0], buf0, sem0).start()
            pltpu.make_async_copy(seq_hbm.at[idx1], buf1, sem1).start()

            def pair(g2, _):
                pltpu.make_async_copy(seq_hbm.at[idx0], buf0, sem0).wait()
                compute_chunk(buf0, b_loc, (2 * g2) * CH)
                bump_idx(idx0)

                @pl.when(g2 < n_ch // 2 - 1)
                def _more0():
                    pltpu.make_async_copy(seq_hbm.at[idx0], buf0, sem0).start()

                pltpu.make_async_copy(seq_hbm.at[idx1], buf1, sem1).wait()
                compute_chunk(buf1, b_loc, (2 * g2 + 1) * CH)
                bump_idx(idx1)

                @pl.when(g2 < n_ch // 2 - 1)
                def _more1():
                    pltpu.make_async_copy(seq_hbm.at[idx1], buf1, sem1).start()

                return 0

            lax.fori_loop(0, n_ch // 2, pair, 0, unroll=False)

        pltpu.sync_copy(sc_v, out_hbm.at[pl.ds(base_b, b_per_w)])

    return scorer(target_vec, seq_flat)


def _merge_body(s_tc, vals_ref, sidx_ref, sc_ref, out_ref):
    b, s_sc = sc_ref.shape
    x = sc_ref[...]
    lane = lax.broadcasted_iota(jnp.int32, (b, s_sc), 1)
    big = jnp.int32(2**30)
    vs, ids = [], []
    for _ in range(NN):
        m = jnp.max(x, axis=1)
        sel = jnp.min(jnp.where(x == m[:, None], lane, big), axis=1)
        x = jnp.where(lane == sel[:, None], -jnp.inf, x)
        vs.append(m)
        ids.append(sel + s_tc)
    sc_vals = jnp.stack(vs, axis=0)
    sc_idx = jnp.stack(ids, axis=0)
    xall = jnp.concatenate([vals_ref[...], sc_vals], axis=0)
    xiall = jnp.concatenate([sidx_ref[...], sc_idx], axis=0)
    _, top_i = _merge_top4(xall, xiall)
    col = lax.broadcasted_iota(jnp.int32, (NN, b), 1)
    out_ref[...] = top_i * b + col


def _tc_merge(vals, sidx, sc_scores, s_tc):
    B, s_sc = sc_scores.shape
    return pl.pallas_call(
        functools.partial(_merge_body, s_tc),
        out_shape=jax.ShapeDtypeStruct((NN, B), jnp.int32),
    )(vals, sidx, sc_scores)


def _sc_gather(table, flat_idx, n_rows, d):
    info = plsc.get_sparse_core_info()
    nw = info.num_cores * info.num_subcores
    per_w = n_rows // nw
    mesh = plsc.VectorSubcoreMesh(core_axis_name="c", subcore_axis_name="s")

    @functools.partial(
        pl.kernel,
        out_type=jax.ShapeDtypeStruct((n_rows, d), jnp.float32),
        mesh=mesh,
        scratch_types=[
            pltpu.VMEM((per_w,), jnp.int32),
            pltpu.VMEM((per_w, d), jnp.float32),
            pltpu.SemaphoreType.DMA,
        ],
    )
    def gather_kernel(table_hbm, idx_hbm, out_hbm, idx_v, rows_v, sem):
        wid = lax.axis_index("s") * info.num_cores + lax.axis_index("c")
        base = wid * per_w
        pltpu.sync_copy(idx_hbm.at[pl.ds(base, per_w)], idx_v)
        pltpu.async_copy(table_hbm.at[idx_v], rows_v, sem).wait()
        pltpu.sync_copy(rows_v, out_hbm.at[pl.ds(base, per_w)])

    return gather_kernel(table, flat_idx)


def kernel(target_vec, seq_vecs):
    S, B, D = seq_vecs.shape
    s_tc = S - S_SC
    flat = seq_vecs.reshape(S * B, D)
    vals, sidx = _tc_topk(target_vec, seq_vecs, s_tc)
    sc_scores = _sc_scores(target_vec, flat, s_tc, S_SC, B, D)
    flat_idx = _tc_merge(vals, sidx, sc_scores, s_tc).reshape(-1)
    rows = _sc_gather(flat, flat_idx, NN * B, D)
    return rows.reshape(NN, B, D)


# speedup vs baseline: 1.1224x; 1.1070x over previous
import functools

import jax
import jax.numpy as jnp
from jax import lax
from jax.experimental import pallas as pl
from jax.experimental.pallas import tpu as pltpu
from jax.experimental.pallas import tpu_sc as plsc

NN = 4


def _merge_top4(x, xi):
    nrows, b = x.shape
    big = jnp.int32(2**30)
    vals, idx = [], []
    for _ in range(NN):
        m = jnp.max(x, axis=0)
        sel = jnp.min(jnp.where(x == m[None], xi, big), axis=0)
        x = jnp.where(xi == sel[None], -jnp.inf, x)
        vals.append(m)
        idx.append(sel)
    return jnp.stack(vals, axis=0), jnp.stack(idx, axis=0)


def _tc_topk_body(num_steps, tgt_ref, seq_ref, idx_out_ref, vals_ref, gidx_ref):
    step = pl.program_id(0)
    sb, b, d = seq_ref.shape

    @pl.when(step == 0)
    def _init():
        vals_ref[...] = jnp.full((NN, b), -jnp.inf, jnp.float32)
        gidx_ref[...] = jnp.zeros((NN, b), jnp.int32)

    seq = seq_ref[...]
    tgt = tgt_ref[...]
    scores = jnp.sum(seq * tgt[None], axis=-1)
    rowid = step * sb + lax.broadcasted_iota(jnp.int32, (sb, b), 0)
    x = jnp.concatenate([vals_ref[...], scores], axis=0)
    xi = jnp.concatenate([gidx_ref[...], rowid], axis=0)
    nv, ni = _merge_top4(x, xi)
    vals_ref[...] = nv
    gidx_ref[...] = ni

    @pl.when(step == num_steps - 1)
    def _fin():
        col = lax.broadcasted_iota(jnp.int32, (NN, b), 1)
        idx_out_ref[...] = gidx_ref[...] * b + col


def _topk_indices(target_vec, seq_vecs, block_s=64):
    S, B, D = seq_vecs.shape
    num_steps = S // block_s
    return pl.pallas_call(
        functools.partial(_tc_topk_body, num_steps),
        grid=(num_steps,),
        in_specs=[
            pl.BlockSpec((B, D), lambda i: (0, 0)),
            pl.BlockSpec((block_s, B, D), lambda i: (i, 0, 0)),
        ],
        out_specs=pl.BlockSpec((NN, B), lambda i: (0, 0)),
        out_shape=jax.ShapeDtypeStruct((NN, B), jnp.int32),
        scratch_shapes=[
            pltpu.VMEM((NN, B), jnp.float32),
            pltpu.VMEM((NN, B), jnp.int32),
        ],
    )(target_vec, seq_vecs)


def _sc_gather(table, flat_idx, n_rows, d):
    info = plsc.get_sparse_core_info()
    nw = info.num_cores * info.num_subcores
    per_w = n_rows // nw
    mesh = plsc.VectorSubcoreMesh(core_axis_name="c", subcore_axis_name="s")

    @functools.partial(
        pl.kernel,
        out_type=jax.ShapeDtypeStruct((n_rows, d), jnp.float32),
        mesh=mesh,
        scratch_types=[
            pltpu.VMEM((per_w,), jnp.int32),
            pltpu.VMEM((per_w, d), jnp.float32),
            pltpu.SemaphoreType.DMA,
        ],
    )
    def gather_kernel(table_hbm, idx_hbm, out_hbm, idx_v, rows_v, sem):
        wid = lax.axis_index("s") * info.num_cores + lax.axis_index("c")
        base = wid * per_w
        pltpu.sync_copy(idx_hbm.at[pl.ds(base, per_w)], idx_v)
        pltpu.async_copy(table_hbm.at[idx_v], rows_v, sem).wait()
        pltpu.sync_copy(rows_v, out_hbm.at[pl.ds(base, per_w)])

    return gather_kernel(table, flat_idx)


def kernel(target_vec, seq_vecs):
    S, B, D = seq_vecs.shape
    flat_idx = _topk_indices(target_vec, seq_vecs).reshape(-1)
    flat = seq_vecs.reshape(S * B, D)
    rows = _sc_gather(flat, flat_idx, NN * B, D)
    return rows.reshape(NN, B, D)


# speedup vs baseline: 1.1261x; 1.0033x over previous
import functools

import jax
import jax.numpy as jnp
from jax import lax
from jax.experimental import pallas as pl
from jax.experimental.pallas import tpu as pltpu
from jax.experimental.pallas import tpu_sc as plsc

NN = 4


def _merge_top4(x, xi):
    nrows, b = x.shape
    big = jnp.int32(2**30)
    vals, idx = [], []
    for _ in range(NN):
        m = jnp.max(x, axis=0)
        sel = jnp.min(jnp.where(x == m[None], xi, big), axis=0)
        x = jnp.where(xi == sel[None], -jnp.inf, x)
        vals.append(m)
        idx.append(sel)
    return jnp.stack(vals, axis=0), jnp.stack(idx, axis=0)


def _tc_topk_body(num_steps, block_s, tgt_ref, seq_ref, idx_out_ref,
                  vals_ref, gidx_ref):
    step = pl.program_id(0)
    rows, d = seq_ref.shape
    b = rows // block_s

    @pl.when(step == 0)
    def _init():
        vals_ref[...] = jnp.full((NN, b), -jnp.inf, jnp.float32)
        gidx_ref[...] = jnp.zeros((NN, b), jnp.int32)

    seq = seq_ref[...].reshape(block_s, b, d)
    tgt = tgt_ref[...]
    scores = jnp.sum(seq * tgt[None], axis=-1)
    rowid = step * block_s + lax.broadcasted_iota(jnp.int32, (block_s, b), 0)
    x = jnp.concatenate([vals_ref[...], scores], axis=0)
    xi = jnp.concatenate([gidx_ref[...], rowid], axis=0)
    nv, ni = _merge_top4(x, xi)
    vals_ref[...] = nv
    gidx_ref[...] = ni

    @pl.when(step == num_steps - 1)
    def _fin():
        col = lax.broadcasted_iota(jnp.int32, (NN, b), 1)
        idx_out_ref[...] = gidx_ref[...] * b + col


def _topk_indices(target_vec, seq_flat, B, block_s=64):
    SB, D = seq_flat.shape
    num_steps = SB // (block_s * B)
    return pl.pallas_call(
        functools.partial(_tc_topk_body, num_steps, block_s),
        grid=(num_steps,),
        in_specs=[
            pl.BlockSpec((B, D), lambda i: (0, 0)),
            pl.BlockSpec((block_s * B, D), lambda i: (i, 0)),
        ],
        out_specs=pl.BlockSpec((NN, B), lambda i: (0, 0)),
        out_shape=jax.ShapeDtypeStruct((NN, B), jnp.int32),
        scratch_shapes=[
            pltpu.VMEM((NN, B), jnp.float32),
            pltpu.VMEM((NN, B), jnp.int32),
        ],
    )(target_vec, seq_flat)


def _sc_gather(table, flat_idx, n_rows, d):
    info = plsc.get_sparse_core_info()
    nw = info.num_cores * info.num_subcores
    per_w = n_rows // nw
    mesh = plsc.VectorSubcoreMesh(core_axis_name="c", subcore_axis_name="s")

    @functools.partial(
        pl.kernel,
        out_type=jax.ShapeDtypeStruct((n_rows, d), jnp.float32),
        mesh=mesh,
        scratch_types=[
            pltpu.VMEM((per_w,), jnp.int32),
            pltpu.VMEM((per_w, d), jnp.float32),
            pltpu.SemaphoreType.DMA,
        ],
    )
    def gather_kernel(table_hbm, idx_hbm, out_hbm, idx_v, rows_v, sem):
        wid = lax.axis_index("s") * info.num_cores + lax.axis_index("c")
        base = wid * per_w
        pltpu.sync_copy(idx_hbm.at[pl.ds(base, per_w)], idx_v)
        pltpu.async_copy(table_hbm.at[idx_v], rows_v, sem).wait()
        pltpu.sync_copy(rows_v, out_hbm.at[pl.ds(base, per_w)])

    return gather_kernel(table, flat_idx)


def kernel(target_vec, seq_vecs):
    S, B, D = seq_vecs.shape
    flat = seq_vecs.reshape(S * B, D)
    flat_idx = _topk_indices(target_vec, flat, B).reshape(-1)
    rows = _sc_gather(flat, flat_idx, NN * B, D)
    return rows.reshape(NN, B, D)
